# baseline (speedup 1.0000x reference)
import functools

import jax
import jax.numpy as jnp
from jax import lax
from jax.experimental import pallas as pl
from jax.experimental.pallas import tpu as pltpu
from jax.experimental.pallas import tpu_sc as plsc

D = 128
LANES = 16
NC, NS = 2, 16
NW = NC * NS
CHUNK = 128
GRP = 8


def _sc_partial(x, srcp, dstp, wp, cpw, n_acc):
    rows_per_tile = n_acc // NS
    ngr = cpw // GRP
    mesh = plsc.VectorSubcoreMesh(core_axis_name="c", subcore_axis_name="s")

    @functools.partial(
        pl.kernel,
        out_type=jax.ShapeDtypeStruct((NC, n_acc, D), jnp.float32),
        mesh=mesh,
        scratch_types=[
            pltpu.VMEM(((cpw + 2) * CHUNK,), jnp.int32),
            pltpu.VMEM((2 * GRP, CHUNK), jnp.int32),
            pltpu.VMEM((2 * GRP * CHUNK,), jnp.float32),
            pltpu.VMEM((CHUNK, D), jnp.float32),
            pltpu.VMEM((CHUNK, D), jnp.float32),
            pltpu.VMEM_SHARED((n_acc, D), jnp.float32),
            pltpu.SemaphoreType.DMA,
            pltpu.SemaphoreType.DMA,
            pltpu.SemaphoreType.DMA,
        ],
        compiler_params=pltpu.CompilerParams(needs_layout_passes=False),
    )
    def k(x_hbm, src_hbm, dst_hbm, w_hbm, part_hbm,
          sidx_v, didx_b, w_b, rows0, rows1, acc_sh, sem0, sem1, semi):
        c = lax.axis_index("c")
        s = lax.axis_index("s")
        wid = c * NS + s

        pltpu.sync_copy(src_hbm.at[wid], sidx_v)

        def fire_idx(g):
            p = lax.rem(g, 2)
            pltpu.async_copy(dst_hbm.at[wid, pl.ds(g * GRP, GRP)],
                             didx_b.at[pl.ds(p * GRP, GRP)], semi)
            pltpu.async_copy(w_hbm.at[wid, pl.ds(g * GRP * CHUNK, GRP * CHUNK)],
                             w_b.at[pl.ds(p * GRP * CHUNK, GRP * CHUNK)], semi)

        def drain_idx(g):
            p = lax.rem(g, 2)
            pltpu.make_async_copy(dst_hbm.at[wid, pl.ds(g * GRP, GRP)],
                                  didx_b.at[pl.ds(p * GRP, GRP)], semi).wait()
            pltpu.make_async_copy(w_hbm.at[wid, pl.ds(g * GRP * CHUNK, GRP * CHUNK)],
                                  w_b.at[pl.ds(p * GRP * CHUNK, GRP * CHUNK)],
                                  semi).wait()

        fire_idx(0)

        def zfill(r, _):
            for j in range(D // LANES):
                rows0[r, pl.ds(j * LANES, LANES)] = jnp.zeros((LANES,), jnp.float32)
            return 0
        lax.fori_loop(0, CHUNK, zfill, 0)

        def zcopy(kk, _):
            pltpu.sync_copy(
                rows0, acc_sh.at[pl.ds(s * rows_per_tile + kk * CHUNK, CHUNK)])
            return 0
        lax.fori_loop(0, rows_per_tile // CHUNK, zcopy, 0)
        plsc.subcore_barrier()

        def sidx(ci):
            return sidx_v.at[pl.ds(ci * CHUNK, CHUNK)]

        def process(buf, lc):
            wbase = lc * CHUNK

            def scale4(q, _):
                for u in range(4):
                    e = q * 4 + u
                    wspl = plsc.load_gather(
                        w_b, [jnp.full((LANES,), wbase + e, jnp.int32)])
                    for j in range(D // LANES):
                        sl = pl.ds(j * LANES, LANES)
                        buf[e, sl] = buf[e, sl] * wspl
                return 0
            lax.fori_loop(0, CHUNK // 4, scale4, 0)
            pltpu.sync_copy(buf, acc_sh.at[didx_b.at[lc]], add=True)

        pltpu.async_copy(x_hbm.at[sidx(0)], rows0, sem0)
        pltpu.async_copy(x_hbm.at[sidx(1)], rows1, sem1)

        def group(g, _):
            p = lax.rem(g, 2)
            drain_idx(g)
            fire_idx(g + 1)

            def pair(j, _):
                c0 = g * GRP + 2 * j
                l0 = p * GRP + 2 * j
                pltpu.make_async_copy(x_hbm.at[sidx(c0)], rows0, sem0).wait()
                process(rows0, l0)
                pltpu.make_async_copy(x_hbm.at[sidx(c0 + 1)], rows1, sem1).wait()
                pltpu.async_copy(x_hbm.at[sidx(c0 + 2)], rows0, sem0)
                process(rows1, l0 + 1)
                pltpu.async_copy(x_hbm.at[sidx(c0 + 3)], rows1, sem1)
                return 0
            lax.fori_loop(0, GRP // 2, pair, 0)
            return 0
        lax.fori_loop(0, ngr, group, 0)

        pltpu.make_async_copy(x_hbm.at[sidx(cpw)], rows0, sem0).wait()
        pltpu.make_async_copy(x_hbm.at[sidx(cpw + 1)], rows1, sem1).wait()
        drain_idx(ngr)
        plsc.subcore_barrier()

        pltpu.sync_copy(
            acc_sh.at[pl.ds(s * rows_per_tile, rows_per_tile)],
            part_hbm.at[c, pl.ds(s * rows_per_tile, rows_per_tile)])

    return k(x, srcp, dstp, wp)


def _finish_body(p0_ref, p1_ref, b_ref, o_ref):
    o_ref[...] = jnp.maximum(p0_ref[...] + p1_ref[...] + b_ref[...], 0.0)


def kernel(x, edge_index, w, bias):
    n_nodes = x.shape[0]
    e = w.shape[0]
    n_chunks = -(-e // CHUNK)
    cpw = -(-n_chunks // NW)
    cpw = -(-cpw // (2 * GRP)) * (2 * GRP)
    e_pad = cpw * NW * CHUNK
    pad = e_pad - e

    src = jnp.pad(edge_index[0], (0, pad)).reshape(NW, cpw * CHUNK)
    src = jnp.pad(src, ((0, 0), (0, 2 * CHUNK)))
    dst = jnp.pad(edge_index[1], (0, pad)).reshape(NW, cpw, CHUNK)
    dst = jnp.pad(dst, ((0, 0), (0, 2 * GRP), (0, 0)))
    wp = jnp.pad(w, (0, pad)).reshape(NW, cpw * CHUNK)
    wp = jnp.pad(wp, ((0, 0), (0, 2 * GRP * CHUNK)))

    n_acc = -(-n_nodes // (NS * CHUNK)) * (NS * CHUNK)
    partial = _sc_partial(x, src, dst, wp, cpw, n_acc)

    blk = 1000
    grid = n_nodes // blk
    out = pl.pallas_call(
        _finish_body,
        grid=(grid,),
        in_specs=[
            pl.BlockSpec((blk, D), lambda i: (i, 0)),
            pl.BlockSpec((blk, D), lambda i: (i, 0)),
            pl.BlockSpec((blk, 1), lambda i: (i, 0)),
        ],
        out_specs=pl.BlockSpec((blk, D), lambda i: (i, 0)),
        out_shape=jax.ShapeDtypeStruct((n_nodes, D), jnp.float32),
    )(partial[0, :n_nodes], partial[1, :n_nodes], bias[:, None])
    return out


# speedup vs baseline: 1.2237x; 1.2237x over previous
import functools

import jax
import jax.numpy as jnp
from jax import lax
from jax.experimental import pallas as pl
from jax.experimental.pallas import tpu as pltpu
from jax.experimental.pallas import tpu_sc as plsc

D = 128
LANES = 16
NC, NS = 2, 16
NW = NC * NS
CHUNK = 128
GRP = 8


def _sc_partial(x, srcp, dstp, wp, cpw, n_acc):
    rows_per_tile = n_acc // NS
    ngr = cpw // GRP
    mesh = plsc.VectorSubcoreMesh(core_axis_name="c", subcore_axis_name="s")

    @functools.partial(
        pl.kernel,
        out_type=jax.ShapeDtypeStruct((NC, n_acc, D), jnp.float32),
        mesh=mesh,
        scratch_types=[
            pltpu.VMEM(((cpw + 2) * CHUNK,), jnp.int32),
            pltpu.VMEM((2 * GRP, CHUNK), jnp.int32),
            pltpu.VMEM((2 * GRP * CHUNK,), jnp.float32),
            pltpu.VMEM((CHUNK, D), jnp.float32),
            pltpu.VMEM((CHUNK, D), jnp.float32),
            pltpu.VMEM_SHARED((n_acc, D), jnp.float32),
            pltpu.SemaphoreType.DMA,
            pltpu.SemaphoreType.DMA,
            pltpu.SemaphoreType.DMA,
        ],
        compiler_params=pltpu.CompilerParams(needs_layout_passes=False),
    )
    def k(x_hbm, src_hbm, dst_hbm, w_hbm, part_hbm,
          sidx_v, didx_b, w_b, rows0, rows1, acc_sh, sem0, sem1, semi):
        c = lax.axis_index("c")
        s = lax.axis_index("s")
        wid = c * NS + s

        pltpu.sync_copy(src_hbm.at[wid], sidx_v)

        def fire_idx(g):
            p = lax.rem(g, 2)
            pltpu.async_copy(dst_hbm.at[wid, pl.ds(g * GRP, GRP)],
                             didx_b.at[pl.ds(p * GRP, GRP)], semi)
            pltpu.async_copy(w_hbm.at[wid, pl.ds(g * GRP * CHUNK, GRP * CHUNK)],
                             w_b.at[pl.ds(p * GRP * CHUNK, GRP * CHUNK)], semi)

        def drain_idx(g):
            p = lax.rem(g, 2)
            pltpu.make_async_copy(dst_hbm.at[wid, pl.ds(g * GRP, GRP)],
                                  didx_b.at[pl.ds(p * GRP, GRP)], semi).wait()
            pltpu.make_async_copy(w_hbm.at[wid, pl.ds(g * GRP * CHUNK, GRP * CHUNK)],
                                  w_b.at[pl.ds(p * GRP * CHUNK, GRP * CHUNK)],
                                  semi).wait()

        fire_idx(0)

        def zfill(r, _):
            for j in range(D // LANES):
                rows0[r, pl.ds(j * LANES, LANES)] = jnp.zeros((LANES,), jnp.float32)
            return 0
        lax.fori_loop(0, CHUNK, zfill, 0)

        def zcopy(kk, _):
            pltpu.sync_copy(
                rows0, acc_sh.at[pl.ds(s * rows_per_tile + kk * CHUNK, CHUNK)])
            return 0
        lax.fori_loop(0, rows_per_tile // CHUNK, zcopy, 0)
        plsc.subcore_barrier()

        def sidx(ci):
            return sidx_v.at[pl.ds(ci * CHUNK, CHUNK)]

        def process(buf, lc):
            wbase = lc * CHUNK

            def scale4(q, _):
                for u in range(4):
                    e = q * 4 + u
                    wspl = plsc.load_gather(
                        w_b, [jnp.full((LANES,), wbase + e, jnp.int32)])
                    for j in range(D // LANES):
                        sl = pl.ds(j * LANES, LANES)
                        buf[e, sl] = buf[e, sl] * wspl
                return 0
            lax.fori_loop(0, CHUNK // 4, scale4, 0)
            pltpu.sync_copy(buf, acc_sh.at[didx_b.at[lc]], add=True)

        def group(g, _):
            p = lax.rem(g, 2)
            drain_idx(g)
            fire_idx(g + 1)

            def chunk(j, _):
                ci = g * GRP + j
                pltpu.async_copy(x_hbm.at[sidx(ci)], rows0, sem0).wait()
                process(rows0, p * GRP + j)
                return 0
            lax.fori_loop(0, GRP, chunk, 0)
            return 0
        lax.fori_loop(0, ngr, group, 0)

        drain_idx(ngr)
        plsc.subcore_barrier()

        pltpu.sync_copy(
            acc_sh.at[pl.ds(s * rows_per_tile, rows_per_tile)],
            part_hbm.at[c, pl.ds(s * rows_per_tile, rows_per_tile)])

    return k(x, srcp, dstp, wp)


def _finish_body(p0_ref, p1_ref, b_ref, o_ref):
    o_ref[...] = jnp.maximum(p0_ref[...] + p1_ref[...] + b_ref[...], 0.0)


def kernel(x, edge_index, w, bias):
    n_nodes = x.shape[0]
    e = w.shape[0]
    n_chunks = -(-e // CHUNK)
    cpw = -(-n_chunks // NW)
    cpw = -(-cpw // (2 * GRP)) * (2 * GRP)
    e_pad = cpw * NW * CHUNK
    pad = e_pad - e

    src = jnp.pad(edge_index[0], (0, pad)).reshape(NW, cpw * CHUNK)
    src = jnp.pad(src, ((0, 0), (0, 2 * CHUNK)))
    dst = jnp.pad(edge_index[1], (0, pad)).reshape(NW, cpw, CHUNK)
    dst = jnp.pad(dst, ((0, 0), (0, 2 * GRP), (0, 0)))
    wp = jnp.pad(w, (0, pad)).reshape(NW, cpw * CHUNK)
    wp = jnp.pad(wp, ((0, 0), (0, 2 * GRP * CHUNK)))

    n_acc = -(-n_nodes // (NS * CHUNK)) * (NS * CHUNK)
    partial = _sc_partial(x, src, dst, wp, cpw, n_acc)

    blk = 1000
    grid = n_nodes // blk
    out = pl.pallas_call(
        _finish_body,
        grid=(grid,),
        in_specs=[
            pl.BlockSpec((blk, D), lambda i: (i, 0)),
            pl.BlockSpec((blk, D), lambda i: (i, 0)),
            pl.BlockSpec((blk, 1), lambda i: (i, 0)),
        ],
        out_specs=pl.BlockSpec((blk, D), lambda i: (i, 0)),
        out_shape=jax.ShapeDtypeStruct((n_nodes, D), jnp.float32),
    )(partial[0, :n_nodes], partial[1, :n_nodes], bias[:, None])
    return out
